# Initial kernel scaffold; baseline (speedup 1.0000x reference)
import jax
import jax.numpy as jnp
from jax.experimental import pallas as pl


def kernel(user_emb, item_emb, fc_w, fc_b, edge_index):
    raise NotImplementedError("write your pallas kernel here")


# speedup vs baseline: 4.2569x; 4.2569x over previous
import jax
import jax.numpy as jnp
from jax import lax
from jax.experimental import pallas as pl
from jax.experimental.pallas import tpu as pltpu
from jax.experimental.pallas import tpu_sc as plsc

N = 50000
NP = 50048
D = 64
HD = 32
E = 1_000_000
EP = 1 << 20
IDX_COLS = 128
IDX_ROWS = EP // IDX_COLS
TROWS = IDX_ROWS // 16
SB = 16
NSB = TROWS // SB
OPT = NP // 16
LAYERS = 3

_mesh = plsc.VectorSubcoreMesh(core_axis_name="c", subcore_axis_name="s")


def _segsum_body(tab, gidx, sidx, out, acc, ibufg, ibufs, rbuf, zbuf, gsems):
    c = lax.axis_index("c")
    s = lax.axis_index("s")

    def zrow(i, _):
        zbuf[i, pl.ds(0, 16)] = jnp.zeros((16,), jnp.float32)
        zbuf[i, pl.ds(16, 16)] = jnp.zeros((16,), jnp.float32)
        return 0

    lax.fori_loop(0, 128, zrow, 0)
    zbase = s * OPT

    def zcp(i, _):
        pltpu.sync_copy(zbuf, acc.at[pl.ds(zbase + i * 128, 128)])
        return 0

    lax.fori_loop(0, 24, zcp, 0)
    pltpu.sync_copy(zbuf.at[pl.ds(0, 56)], acc.at[pl.ds(zbase + 3072, 56)])
    plsc.subcore_barrier()

    row0 = s * TROWS

    def sb(k, _):
        r0 = row0 + k * SB
        pltpu.sync_copy(gidx.at[c, pl.ds(r0, SB)], ibufg)
        pltpu.sync_copy(sidx.at[pl.ds(r0, SB)], ibufs)
        for q in range(3):
            pltpu.async_copy(tab.at[ibufg.at[q]], rbuf.at[q], gsems.at[q])

        def quad(tt, _):
            base = 4 * tt
            for q in range(4):
                t = base + q
                nxt = t + 3

                @pl.when(nxt < SB)
                def _():
                    pltpu.async_copy(
                        tab.at[ibufg.at[nxt]], rbuf.at[(3 + q) % 4],
                        gsems.at[(3 + q) % 4])

                pltpu.make_async_copy(
                    tab.at[ibufg.at[t]], rbuf.at[q], gsems.at[q]).wait()
                pltpu.sync_copy(rbuf.at[q], acc.at[ibufs.at[t]], add=True)
            return 0

        lax.fori_loop(0, SB // 4, quad, 0)
        return 0

    lax.fori_loop(0, NSB, sb, 0)
    plsc.subcore_barrier()
    ob = s * OPT
    pltpu.sync_copy(acc.at[pl.ds(ob, OPT)], out.at[pl.ds(c * NP + ob, OPT)])


_segsum = pl.kernel(
    _segsum_body,
    out_type=jax.ShapeDtypeStruct((2 * NP, HD), jnp.float32),
    mesh=_mesh,
    compiler_params=pltpu.CompilerParams(use_tc_tiling_on_sc=False),
    scratch_types=[
        pltpu.VMEM_SHARED((NP, HD), jnp.float32),
        pltpu.VMEM((SB, IDX_COLS), jnp.int32),
        pltpu.VMEM((SB, IDX_COLS), jnp.int32),
        pltpu.VMEM((4, IDX_COLS, HD), jnp.float32),
        pltpu.VMEM((128, HD), jnp.float32),
        pltpu.SemaphoreType.DMA((4,)),
    ],
)


def _deg_body(dsts, srcs, cnt, acc, ibuf, ones, zbuf):
    c = lax.axis_index("c")
    s = lax.axis_index("s")

    def fill(i, _):
        zbuf[i, pl.ds(0, 16)] = jnp.zeros((16,), jnp.float32)
        return 0

    lax.fori_loop(0, 128, fill, 0)

    def fillo(i, _):
        ones[i, pl.ds(0, 16)] = jnp.ones((16,), jnp.float32)
        return 0

    lax.fori_loop(0, IDX_COLS, fillo, 0)
    zbase = s * OPT

    def zcp(i, _):
        pltpu.sync_copy(zbuf, acc.at[pl.ds(zbase + i * 128, 128)])
        return 0

    lax.fori_loop(0, 24, zcp, 0)
    pltpu.sync_copy(zbuf.at[pl.ds(0, 56)], acc.at[pl.ds(zbase + 3072, 56)])
    plsc.subcore_barrier()

    row0 = s * TROWS

    def count(idx):
        def sb(k, _):
            pltpu.sync_copy(idx.at[pl.ds(row0 + k * SB, SB)], ibuf)

            def ch(j, _):
                pltpu.sync_copy(ones, acc.at[ibuf.at[j]], add=True)
                return 0

            lax.fori_loop(0, SB, ch, 0)
            return 0

        lax.fori_loop(0, NSB, sb, 0)

    @pl.when(c == 0)
    def _():
        count(dsts)

    @pl.when(c == 1)
    def _():
        count(srcs)

    plsc.subcore_barrier()
    ob = s * OPT
    pltpu.sync_copy(acc.at[pl.ds(ob, OPT)], cnt.at[c, pl.ds(ob, OPT)])


_deg = pl.kernel(
    _deg_body,
    out_type=jax.ShapeDtypeStruct((2, NP, 16), jnp.float32),
    mesh=_mesh,
    compiler_params=pltpu.CompilerParams(use_tc_tiling_on_sc=False),
    scratch_types=[
        pltpu.VMEM_SHARED((NP, 16), jnp.float32),
        pltpu.VMEM((SB, IDX_COLS), jnp.int32),
        pltpu.VMEM((IDX_COLS, 16), jnp.float32),
        pltpu.VMEM((128, 16), jnp.float32),
    ],
)


R = NP // 16
GB = NP // R


def _fc_body(x_ref, w_ref, b_ref, o_ref):
    c = pl.program_id(0)
    o_ref[...] = (
        lax.dot_general(
            x_ref[...], w_ref[...], (((1,), (1,)), ((), ())),
            preferred_element_type=jnp.float32,
        )
        + b_ref[c, :][None, :]
    )


def _fc(item_pad, fc_w, fc_b):
    b2 = fc_b.reshape(2, HD)
    return pl.pallas_call(
        _fc_body,
        grid=(2, GB),
        in_specs=[
            pl.BlockSpec((R, D), lambda c, g: (g, 0)),
            pl.BlockSpec((HD, D), lambda c, g: (c, 0)),
            pl.BlockSpec((2, HD), lambda c, g: (0, 0)),
        ],
        out_specs=pl.BlockSpec((R, HD), lambda c, g: (c * GB + g, 0)),
        out_shape=jax.ShapeDtypeStruct((2 * NP, HD), jnp.float32),
    )(item_pad, fc_w, b2)


def _norm_body(acca, accb, cnti, cntu, sumi, sumu, hi, hu, soi, sou):
    rii = 1.0 / jnp.maximum(cnti[:, 0:1], 1.0)
    riu = 1.0 / jnp.maximum(cntu[:, 0:1], 1.0)
    new_i = acca[...] * rii
    new_u = accb[...] * riu
    hi[...] = new_i
    hu[...] = new_u
    soi[...] = sumi[...] + new_i
    sou[...] = sumu[...] + new_u


def _norm(acca, accb, cnti, cntu, sumi, sumu):
    flat = pl.BlockSpec((R, HD), lambda g: (g, 0))
    cspec = pl.BlockSpec((R, 16), lambda g: (g % GB, 0))
    return pl.pallas_call(
        _norm_body,
        grid=(2 * GB,),
        in_specs=[flat, flat, cspec, cspec, flat, flat],
        out_specs=[flat, flat, flat, flat],
        out_shape=[jax.ShapeDtypeStruct((2 * NP, HD), jnp.float32)] * 4,
    )(acca, accb, cnti, cntu, sumi, sumu)


def _final_body(a0, a1, b0, b1, cnti, cntu, si0, si1, su0, su1, item_o, user_o):
    rii = 1.0 / jnp.maximum(cnti[:, 0:1], 1.0)
    riu = 1.0 / jnp.maximum(cntu[:, 0:1], 1.0)
    it0 = (si0[...] + a0[...] * rii) * 0.25
    it1 = (si1[...] + a1[...] * rii) * 0.25
    us0 = (su0[...] + b0[...] * riu) * 0.25
    us1 = (su1[...] + b1[...] * riu) * 0.25
    item_o[...] = jnp.concatenate([it0, it1], axis=1)
    user_o[...] = jnp.concatenate([us0, us1], axis=1)


def _final(acca, accb, cnti, cntu, sumi, sumu):
    f0 = pl.BlockSpec((R, HD), lambda g: (g, 0))
    f1 = pl.BlockSpec((R, HD), lambda g: (GB + g, 0))
    cspec = pl.BlockSpec((R, 16), lambda g: (g, 0))
    wide = pl.BlockSpec((R, D), lambda g: (g, 0))
    return pl.pallas_call(
        _final_body,
        grid=(GB,),
        in_specs=[f0, f1, f0, f1, cspec, cspec, f0, f1, f0, f1],
        out_specs=[wide, wide],
        out_shape=[jax.ShapeDtypeStruct((NP, D), jnp.float32)] * 2,
    )(acca, acca, accb, accb, cnti, cntu, sumi, sumi, sumu, sumu)


def _flat_split(x_pad):
    return jnp.concatenate([x_pad[:, :HD], x_pad[:, HD:]], axis=0)


def kernel(user_emb, item_emb, fc_w, fc_b, edge_index):
    src = edge_index[0].astype(jnp.int32)
    dst = edge_index[1].astype(jnp.int32)
    pad = EP - E
    padg = jnp.zeros((pad,), jnp.int32)
    pads = N + (jnp.arange(pad, dtype=jnp.int32) % (NP - N))

    srcg0 = jnp.concatenate([src, padg])
    dstg0 = jnp.concatenate([dst, padg])
    src_g = jnp.stack([srcg0, srcg0 + NP]).reshape(2, IDX_ROWS, IDX_COLS)
    dst_g = jnp.stack([dstg0, dstg0 + NP]).reshape(2, IDX_ROWS, IDX_COLS)
    src_s = jnp.concatenate([src, pads]).reshape(IDX_ROWS, IDX_COLS)
    dst_s = jnp.concatenate([dst, pads]).reshape(IDX_ROWS, IDX_COLS)

    cnt = _deg(dst_s, src_s)
    cnti = cnt[0]
    cntu = cnt[1]

    user_pad = jnp.pad(user_emb, ((0, NP - N), (0, 0)))
    item_pad = jnp.pad(item_emb, ((0, NP - N), (0, 0)))

    hu = _flat_split(user_pad)
    hi = _fc(item_pad, fc_w, fc_b)
    sumu = hu
    sumi = _flat_split(item_pad)

    for layer in range(LAYERS):
        acca = _segsum(hu, src_g, dst_s)
        accb = _segsum(hi, dst_g, src_s)
        if layer < LAYERS - 1:
            hi, hu, sumi, sumu = _norm(acca, accb, cnti, cntu, sumi, sumu)
        else:
            item_out, user_out = _final(acca, accb, cnti, cntu, sumi, sumu)

    return (user_out[:N], item_out[:N])
